# baseline (speedup 1.0000x reference)
import functools

import jax
import jax.numpy as jnp
from jax import lax
from jax.experimental import pallas as pl
from jax.experimental.pallas import tpu as pltpu
from jax.experimental.pallas import tpu_sc as plsc

_NUM_CORES = 2
_NUM_SUBCORES = 16
_NUM_WORKERS = _NUM_CORES * _NUM_SUBCORES


@functools.lru_cache(maxsize=None)
def _build(T: int, S: int, V: int, D: int):
  assert D == 32 and T % (128 * _NUM_WORKERS) == 0
  CC = D // 8
  NT = T // 128
  JPW = NT // _NUM_WORKERS
  CH = 128 * JPW

  mesh = plsc.VectorSubcoreMesh(
      core_axis_name="c", subcore_axis_name="s",
      num_cores=_NUM_CORES, num_subcores=_NUM_SUBCORES)

  @functools.partial(
      pl.kernel,
      out_type=jax.ShapeDtypeStruct((S * CC * NT * 1024,), jnp.float32),
      mesh=mesh,
      scratch_types=[
          [pltpu.VMEM((CH,), jnp.int32)] * 2,
          [pltpu.VMEM((CH, D), jnp.float32)] * 2,
          [pltpu.VMEM((CH * (D + 1),), jnp.float32)] * 2,
          [pltpu.VMEM((JPW * CC * 1024,), jnp.float32)] * 2,
          [pltpu.SemaphoreType.DMA] * 2,
          [pltpu.SemaphoreType.DMA] * 2,
          [pltpu.SemaphoreType.DMA] * 2,
      ],
      compiler_params=pltpu.CompilerParams(
          use_tc_tiling_on_sc=False, needs_layout_passes=False),
  )
  def gather_kernel(tok_hbm, tab_hbm, out_hbm, idx_v, rows_v, rows33_v,
                    trans_v, isem, gsem, osem):
    w = lax.axis_index("s") * _NUM_CORES + lax.axis_index("c")
    iota16 = lax.iota(jnp.int32, 16)
    iota33 = iota16 * (D + 1)

    def idx_copy(s, b):
      return pltpu.make_async_copy(
          tok_hbm.at[pl.ds(s * T + w * CH, CH)], idx_v[b], isem[b])

    def gather(b):
      return pltpu.make_async_copy(
          tab_hbm.at[idx_v[b]], rows_v[b], gsem[b])

    def out_copies(s, b):
      return [pltpu.make_async_copy(
                  trans_v[b].at[pl.ds((j * CC + cc) * 1024, 1024)],
                  out_hbm.at[pl.ds(
                      ((s * CC + cc) * NT + JPW * w + j) * 1024, 1024)],
                  osem[b])
              for j in range(JPW) for cc in range(CC)]

    idx_copy(0, 0).start()
    idx_copy(0, 0).wait()
    gather(0).start()
    idx_copy(1, 1).start()

    def _body(s, b):
      gather(b).wait()

      @pl.when(s + 2 < S)
      def _():
        idx_copy(s + 2, b).start()

      @pl.when(s + 1 < S)
      def _():
        idx_copy(s + 1, 1 - b).wait()
        gather(1 - b).start()

      @pl.when(s >= 2)
      def _():
        for c in out_copies(s - 2, b):
          c.wait()

      @pl.loop(0, CH, unroll=32)
      def _pad(r):
        rows33_v[b][pl.ds(r * (D + 1), 16)] = rows_v[b][r, pl.ds(0, 16)]
        rows33_v[b][pl.ds(r * (D + 1) + 16, 16)] = (
            rows_v[b][r, pl.ds(16, 16)])

      @pl.loop(0, D, unroll=8)
      def _chan(c):
        g = (c // 8) * 1024 + lax.rem(c, 8) * 128
        for j in range(JPW):
          jb = j * (CC * 1024)
          for k in range(8):
            src = iota33 + ((j * 128 + 16 * k) * (D + 1) + c)
            v = plsc.load_gather(rows33_v[b], [src])
            trans_v[b][pl.ds(jb + g + 16 * k, 16)] = v

      for cpy in out_copies(s, b):
        cpy.start()

    @pl.loop(0, S)
    def _chunk(s):
      b = lax.rem(s, 2)

      @pl.when(b == 0)
      def _():
        _body(s, 0)

      @pl.when(b == 1)
      def _():
        _body(s, 1)

    for c in out_copies(S - 2, (S - 2) % 2):
      c.wait()
    for c in out_copies(S - 1, (S - 1) % 2):
      c.wait()

  return gather_kernel


def kernel(token_ids, embeddings):
  T, S = token_ids.shape
  V, D = embeddings.shape
  tok_sm = jnp.reshape(jnp.transpose(token_ids.astype(jnp.int32)), (-1,))
  out_flat = _build(T, S, V, D)(tok_sm, embeddings)
  out5 = jnp.reshape(out_flat, (S, D // 8, T // 128, 8, 128))
  out = jnp.reshape(jnp.transpose(out5, (2, 4, 0, 1, 3)), (T, S, D))
  return out


# speedup vs baseline: 1.0103x; 1.0103x over previous
import functools

import jax
import jax.numpy as jnp
from jax import lax
from jax.experimental import pallas as pl
from jax.experimental.pallas import tpu as pltpu
from jax.experimental.pallas import tpu_sc as plsc

_NUM_CORES = 2
_NUM_SUBCORES = 16
_NUM_WORKERS = _NUM_CORES * _NUM_SUBCORES


@functools.lru_cache(maxsize=None)
def _build(T: int, S: int, V: int, D: int):
  assert D == 32 and T % (128 * _NUM_WORKERS) == 0
  CC = D // 8
  NT = T // 128
  JPW = NT // _NUM_WORKERS
  CH = 128 * JPW

  mesh = plsc.VectorSubcoreMesh(
      core_axis_name="c", subcore_axis_name="s",
      num_cores=_NUM_CORES, num_subcores=_NUM_SUBCORES)

  @functools.partial(
      pl.kernel,
      out_type=jax.ShapeDtypeStruct((S * CC * NT * 1024,), jnp.float32),
      mesh=mesh,
      scratch_types=[
          [pltpu.VMEM((CH,), jnp.int32)] * 2,
          [pltpu.VMEM((CH, D), jnp.float32)] * 2,
          [pltpu.VMEM((CH * (D + 1),), jnp.float32)] * 2,
          [pltpu.VMEM((JPW * CC * 1024,), jnp.float32)] * 2,
          [pltpu.SemaphoreType.DMA] * 2,
          [pltpu.SemaphoreType.DMA] * 2,
          [pltpu.SemaphoreType.DMA] * 2,
      ],
      compiler_params=pltpu.CompilerParams(
          use_tc_tiling_on_sc=False, needs_layout_passes=False),
  )
  def gather_kernel(tok_hbm, tab_hbm, out_hbm, idx_v, rows_v, rows33_v,
                    trans_v, isem, gsem, osem):
    w = lax.axis_index("s") * _NUM_CORES + lax.axis_index("c")
    iota16 = lax.iota(jnp.int32, 16)
    iota33 = iota16 * (D + 1)

    def idx_copy(s, b):
      return pltpu.make_async_copy(
          tok_hbm.at[pl.ds(s * T + w * CH, CH)], idx_v[b], isem[b])

    def gather(b):
      return pltpu.make_async_copy(
          tab_hbm.at[idx_v[b]], rows_v[b], gsem[b])

    def out_copies(s, b):
      return [pltpu.make_async_copy(
                  trans_v[b].at[pl.ds((j * CC + cc) * 1024, 1024)],
                  out_hbm.at[pl.ds(
                      ((s * CC + cc) * NT + JPW * w + j) * 1024, 1024)],
                  osem[b])
              for j in range(JPW) for cc in range(CC)]

    idx_copy(0, 0).start()
    idx_copy(0, 0).wait()
    gather(0).start()
    idx_copy(1, 1).start()

    def _body(s, b):
      gather(b).wait()

      @pl.when(s + 2 < S)
      def _():
        idx_copy(s + 2, b).start()

      @pl.when(s + 1 < S)
      def _():
        idx_copy(s + 1, 1 - b).wait()
        gather(1 - b).start()

      @pl.when(s >= 2)
      def _():
        for c in out_copies(s - 2, b):
          c.wait()

      @pl.loop(0, CH, unroll=16)
      def _pad(r):
        rows33_v[b][pl.ds(r * (D + 1), 16)] = rows_v[b][r, pl.ds(0, 16)]
        rows33_v[b][pl.ds(r * (D + 1) + 16, 16)] = (
            rows_v[b][r, pl.ds(16, 16)])

      @pl.loop(0, D, unroll=4)
      def _chan(c):
        g = (c // 8) * 1024 + lax.rem(c, 8) * 128
        for j in range(JPW):
          jb = j * (CC * 1024)
          for k in range(8):
            src = iota33 + ((j * 128 + 16 * k) * (D + 1) + c)
            v = plsc.load_gather(rows33_v[b], [src])
            trans_v[b][pl.ds(jb + g + 16 * k, 16)] = v

      for cpy in out_copies(s, b):
        cpy.start()

    @pl.loop(0, S)
    def _chunk(s):
      b = lax.rem(s, 2)

      @pl.when(b == 0)
      def _():
        _body(s, 0)

      @pl.when(b == 1)
      def _():
        _body(s, 1)

    for c in out_copies(S - 2, (S - 2) % 2):
      c.wait()
    for c in out_copies(S - 1, (S - 1) % 2):
      c.wait()

  return gather_kernel


def kernel(token_ids, embeddings):
  T, S = token_ids.shape
  V, D = embeddings.shape
  tok_sm = jnp.reshape(jnp.transpose(token_ids.astype(jnp.int32)), (-1,))
  out_flat = _build(T, S, V, D)(tok_sm, embeddings)
  out5 = jnp.reshape(out_flat, (S, D // 8, T // 128, 8, 128))
  out = jnp.reshape(jnp.transpose(out5, (2, 4, 0, 1, 3)), (T, S, D))
  return out
